# baseline (speedup 1.0000x reference)
import functools
import math

import jax
import jax.numpy as jnp
from jax.experimental import pallas as pl

_SQRT_2_OVER_PI = math.sqrt(2.0 / math.pi)

_SORT8 = [(0, 1), (2, 3), (4, 5), (6, 7),
          (0, 2), (1, 3), (4, 6), (5, 7),
          (1, 2), (5, 6), (0, 4), (3, 7),
          (1, 5), (2, 6),
          (1, 4), (3, 6),
          (2, 4), (3, 5),
          (3, 4)]


def _gelu(x):
    return 0.5 * x * (1.0 + jnp.tanh(_SQRT_2_OVER_PI * (x + 0.044715 * x * x * x)))


def _colsum(v):
    ones = jnp.ones((1, v.shape[0]), jnp.float32)
    return jax.lax.dot_general(ones, v, (((1,), (0,)), ((), ())),
                               preferred_element_type=jnp.float32)


def _stats_body(x_ref, s1_ref, s2_ref, s3_ref):
    i = pl.program_id(0)
    xb = x_ref[...]
    g = _gelu(xb)
    s1 = _colsum(xb)
    s2 = _colsum(xb * xb)
    s3 = _colsum(g)

    @pl.when(i == 0)
    def _init():
        s1_ref[...] = s1
        s2_ref[...] = s2
        s3_ref[...] = s3

    @pl.when(i != 0)
    def _acc():
        s1_ref[...] += s1
        s2_ref[...] += s2
        s3_ref[...] += s3


def _gate_body(k, rc, x_ref, p_ref, o_ref):
    mu = p_ref[0:1, :]
    rstd = p_ref[1:2, :]
    ema = p_ref[2:3, :]
    tau = p_ref[3:4, :]
    beta_up = p_ref[4:5, :]
    gamma = p_ref[5:6, :]
    beta_fam = p_ref[6:7, :]
    rb = x_ref.shape[0]

    def chunk(c, _):
        r = c * rc
        xb = x_ref[pl.ds(r, rc), :]
        z = (xb - mu) * rstd
        a = jnp.abs(z)

        d = a.shape[-1]
        w = d // 8
        parts = [jax.lax.slice_in_dim(a, j * w, (j + 1) * w, axis=1)
                 for j in range(8)]
        for i, j in _SORT8:
            hi = jnp.maximum(parts[i], parts[j])
            lo = jnp.minimum(parts[i], parts[j])
            parts[i], parts[j] = hi, lo

        tops = list(parts)
        for it in range(k):
            t_top = jnp.max(tops[0], axis=-1, keepdims=True)
            if it < k - 1:
                depth = min(8, k - 1 - it)
                mask = tops[0] >= t_top
                for j in range(depth - 1):
                    tops[j] = jnp.where(mask, tops[j + 1], tops[j])
                tops[depth - 1] = jnp.where(
                    mask,
                    tops[depth] if depth < 8 else jnp.float32(-jnp.inf),
                    tops[depth - 1])

        bots = list(parts)
        for it in range(k):
            t_bot = jnp.min(bots[7], axis=-1, keepdims=True)
            if it < k - 1:
                depth = min(8, k - 1 - it)
                mask = bots[7] <= t_bot
                for j in range(depth - 1):
                    bots[7 - j] = jnp.where(mask, bots[6 - j], bots[7 - j])
                bots[8 - depth] = jnp.where(
                    mask,
                    bots[7 - depth] if depth < 8 else jnp.float32(jnp.inf),
                    bots[8 - depth])

        out = _gelu(xb)
        emadot = jax.lax.dot_general(
            out, jnp.transpose(ema), (((1,), (0,)), ((), ())),
            preferred_element_type=jnp.float32)
        out2 = out * out
        nrm2 = jax.lax.dot_general(
            out2, jnp.ones((d, 1), jnp.float32), (((1,), (0,)), ((), ())),
            preferred_element_type=jnp.float32)
        nrm = jnp.maximum(jnp.sqrt(nrm2), 1e-12)
        cos = jnp.clip(emadot / nrm, -1.0, 1.0)
        gcos = jnp.exp(-cos * tau[0:1, 0:1])

        gtop = jnp.clip(1.0 + beta_up * jnp.tanh(gamma * z), 0.1, 8.0)
        gate = jnp.where(a >= t_top, gtop, jnp.float32(1.0))
        gate = jnp.where(a <= t_bot, beta_fam, gate)
        o_ref[pl.ds(r, rc), :] = out * gate * gcos
        return _

    for c in range(rb // rc):
        chunk(c, 0)


def kernel(x, logit_decay, log_tau, log_beta_up, log_gamma, logit_beta_fam):
    B, T, D = x.shape
    N = B * T
    k = min(16, D // 2)
    xf = x.reshape(N, D)

    ra = 2048
    while N % ra:
        ra //= 2
    s1, s2, s3 = pl.pallas_call(
        _stats_body,
        grid=(N // ra,),
        in_specs=[pl.BlockSpec((ra, D), lambda i: (i, 0))],
        out_specs=[pl.BlockSpec((1, D), lambda i: (0, 0))] * 3,
        out_shape=[jax.ShapeDtypeStruct((1, D), jnp.float32)] * 3,
    )(xf)

    inv_n = jnp.float32(1.0 / N)
    mean = s1[0] * inv_n
    mean_sq = s2[0] * inv_n
    mean_out = s3[0] * inv_n
    var = jnp.maximum(mean_sq - mean * mean, 1e-4)
    rstd = 1.0 / (jnp.sqrt(var) + 1e-5)
    ema_n = mean_out / jnp.maximum(jnp.linalg.norm(mean_out), 1e-12)
    tau = jnp.exp(log_tau)
    beta_up = jax.nn.softplus(log_beta_up)
    gamma = jax.nn.softplus(log_gamma)
    beta_fam = jax.nn.sigmoid(logit_beta_fam)
    ones = jnp.ones((D,), jnp.float32)
    params = jnp.stack(
        [mean, rstd, ema_n, tau * ones, beta_up * ones, gamma * ones,
         beta_fam * ones, jnp.zeros((D,), jnp.float32)])

    rb = 512
    while N % rb:
        rb //= 2
    rc = 64 if rb % 64 == 0 else rb
    out = pl.pallas_call(
        functools.partial(_gate_body, k, rc),
        grid=(N // rb,),
        in_specs=[
            pl.BlockSpec((rb, D), lambda i: (i, 0)),
            pl.BlockSpec((8, D), lambda i: (0, 0)),
        ],
        out_specs=pl.BlockSpec((rb, D), lambda i: (i, 0)),
        out_shape=jax.ShapeDtypeStruct((N, D), jnp.float32),
    )(xf, params)
    return out.reshape(B, T, D)


# speedup vs baseline: 1.4486x; 1.4486x over previous
import functools
import math

import jax
import jax.numpy as jnp
from jax.experimental import pallas as pl

_SQRT_2_OVER_PI = math.sqrt(2.0 / math.pi)

_SORT8 = [(0, 1), (2, 3), (4, 5), (6, 7),
          (0, 2), (1, 3), (4, 6), (5, 7),
          (1, 2), (5, 6), (0, 4), (3, 7),
          (1, 5), (2, 6),
          (1, 4), (3, 6),
          (2, 4), (3, 5),
          (3, 4)]


def _gelu(x):
    return 0.5 * x * (1.0 + jnp.tanh(_SQRT_2_OVER_PI * (x + 0.044715 * x * x * x)))


def _colsum(v):
    ones = jnp.ones((1, v.shape[0]), jnp.float32)
    return jax.lax.dot_general(ones, v, (((1,), (0,)), ((), ())),
                               preferred_element_type=jnp.float32)


def _stats_body(x_ref, s1_ref, s2_ref, s3_ref):
    i = pl.program_id(0)
    xb = x_ref[...]
    g = _gelu(xb)
    s1 = _colsum(xb)
    s2 = _colsum(xb * xb)
    s3 = _colsum(g)

    @pl.when(i == 0)
    def _init():
        s1_ref[...] = s1
        s2_ref[...] = s2
        s3_ref[...] = s3

    @pl.when(i != 0)
    def _acc():
        s1_ref[...] += s1
        s2_ref[...] += s2
        s3_ref[...] += s3


def _gate_body(k, rc, x_ref, p_ref, o_ref):
    mu = p_ref[0:1, :]
    rstd = p_ref[1:2, :]
    ema = p_ref[2:3, :]
    tau = p_ref[3:4, :]
    beta_up = p_ref[4:5, :]
    gamma = p_ref[5:6, :]
    beta_fam = p_ref[6:7, :]
    rb = x_ref.shape[0]

    def chunk(c, _):
        r = c * rc
        xb = x_ref[pl.ds(r, rc), :]
        z = (xb - mu) * rstd
        a = jnp.abs(z)

        d = a.shape[-1]
        w = d // 8
        parts = [jax.lax.slice_in_dim(a, j * w, (j + 1) * w, axis=1)
                 for j in range(8)]
        for i, j in _SORT8:
            hi = jnp.maximum(parts[i], parts[j])
            lo = jnp.minimum(parts[i], parts[j])
            parts[i], parts[j] = hi, lo

        tops = list(parts)
        for it in range(k):
            t_top = jnp.max(tops[0], axis=-1, keepdims=True)
            if it < k - 1:
                depth = min(8, k - 1 - it)
                mask = tops[0] >= t_top
                for j in range(depth - 1):
                    tops[j] = jnp.where(mask, tops[j + 1], tops[j])
                tops[depth - 1] = jnp.where(
                    mask,
                    tops[depth] if depth < 8 else jnp.float32(-jnp.inf),
                    tops[depth - 1])

        bots = list(parts)
        for it in range(k):
            t_bot = jnp.min(bots[7], axis=-1, keepdims=True)
            if it < k - 1:
                depth = min(8, k - 1 - it)
                mask = bots[7] <= t_bot
                for j in range(depth - 1):
                    bots[7 - j] = jnp.where(mask, bots[6 - j], bots[7 - j])
                bots[8 - depth] = jnp.where(
                    mask,
                    bots[7 - depth] if depth < 8 else jnp.float32(jnp.inf),
                    bots[8 - depth])

        out = _gelu(xb)
        emadot = jax.lax.dot_general(
            out, jnp.transpose(ema), (((1,), (0,)), ((), ())),
            preferred_element_type=jnp.float32)
        out2 = out * out
        nrm2 = jax.lax.dot_general(
            out2, jnp.ones((d, 1), jnp.float32), (((1,), (0,)), ((), ())),
            preferred_element_type=jnp.float32)
        nrm = jnp.maximum(jnp.sqrt(nrm2), 1e-12)
        cos = jnp.clip(emadot / nrm, -1.0, 1.0)
        gcos = jnp.exp(-cos * tau[0:1, 0:1])

        gtop = jnp.clip(1.0 + beta_up * jnp.tanh(gamma * z), 0.1, 8.0)
        gate = jnp.where(a >= t_top, gtop, jnp.float32(1.0))
        gate = jnp.where(a <= t_bot, beta_fam, gate)
        o_ref[pl.ds(r, rc), :] = out * gate * gcos
        return _

    for c in range(rb // rc):
        chunk(c, 0)


def kernel(x, logit_decay, log_tau, log_beta_up, log_gamma, logit_beta_fam):
    B, T, D = x.shape
    N = B * T
    k = min(16, D // 2)
    xf = x.reshape(N, D)

    ra = 1024
    while N % ra:
        ra //= 2
    s1, s2, s3 = pl.pallas_call(
        _stats_body,
        grid=(N // ra,),
        in_specs=[pl.BlockSpec((ra, D), lambda i: (i, 0))],
        out_specs=[pl.BlockSpec((1, D), lambda i: (0, 0))] * 3,
        out_shape=[jax.ShapeDtypeStruct((1, D), jnp.float32)] * 3,
    )(xf)

    inv_n = jnp.float32(1.0 / N)
    mean = s1[0] * inv_n
    mean_sq = s2[0] * inv_n
    mean_out = s3[0] * inv_n
    var = jnp.maximum(mean_sq - mean * mean, 1e-4)
    rstd = 1.0 / (jnp.sqrt(var) + 1e-5)
    ema_n = mean_out / jnp.maximum(jnp.linalg.norm(mean_out), 1e-12)
    tau = jnp.exp(log_tau)
    beta_up = jax.nn.softplus(log_beta_up)
    gamma = jax.nn.softplus(log_gamma)
    beta_fam = jax.nn.sigmoid(logit_beta_fam)
    ones = jnp.ones((D,), jnp.float32)
    params = jnp.stack(
        [mean, rstd, ema_n, tau * ones, beta_up * ones, gamma * ones,
         beta_fam * ones, jnp.zeros((D,), jnp.float32)])

    rb = 256
    while N % rb:
        rb //= 2
    rc = 128 if rb % 128 == 0 else rb
    out = pl.pallas_call(
        functools.partial(_gate_body, k, rc),
        grid=(N // rb,),
        in_specs=[
            pl.BlockSpec((rb, D), lambda i: (i, 0)),
            pl.BlockSpec((8, D), lambda i: (0, 0)),
        ],
        out_specs=pl.BlockSpec((rb, D), lambda i: (i, 0)),
        out_shape=jax.ShapeDtypeStruct((N, D), jnp.float32),
    )(xf, params)
    return out.reshape(B, T, D)
